# Initial kernel scaffold; baseline (speedup 1.0000x reference)
import jax
import jax.numpy as jnp
from jax.experimental import pallas as pl


def kernel(x, type_embeddings, atom_types, gate_W, gate_b, expert_W, expert_b):
    raise NotImplementedError("write your pallas kernel here")


# speedup vs baseline: 1.6840x; 1.6840x over previous
import functools

import jax
import jax.numpy as jnp
from jax import lax
from jax.experimental import pallas as pl
from jax.experimental.pallas import tpu as pltpu
from jax.experimental.pallas import tpu_sc as plsc

N_TOK = 8192
NUM_IN = 256
TOTAL_OUT = 256
HALF_OUT = 128
E = 16
NTYPES = 128
TEBD = 64

TM = 128
CAP = 10240
NB = CAP // TM

NC, NS = 2, 16
NW = NC * NS
TPT = N_TOK // NW
RPT = TPT // 128


def _wid():
    return lax.axis_index("s") * NC + lax.axis_index("c")


def _mesh():
    return plsc.VectorSubcoreMesh(core_axis_name="c", subcore_axis_name="s")


_SC_PARAMS = pltpu.CompilerParams(needs_layout_passes=False)


def _emap_body(te_ref, gw_ref, gb_ref, em_ref):
    logits = jnp.dot(te_ref[...], gw_ref[...],
                     preferred_element_type=jnp.float32) + gb_ref[...]
    em_ref[...] = jnp.argmax(logits, axis=1).astype(jnp.int32)[None, :]


def _expert_map(type_embeddings, gate_W, gate_b):
    return pl.pallas_call(
        _emap_body,
        out_shape=jax.ShapeDtypeStruct((1, NTYPES), jnp.int32),
    )(type_embeddings, gate_W, gate_b.reshape(1, E))


def _ids_body(em_hbm, at_hbm, ev_hbm, lc_hbm, em_v, at_v, ev_v, lc_v):
    wid = _wid()
    lane = lax.broadcasted_iota(jnp.int32, (E,), 0)
    pltpu.sync_copy(em_hbm.at[0], em_v)
    pltpu.sync_copy(at_hbm.at[pl.ds(wid * TPT, TPT)], at_v)

    def _hist(i, lc):
        tv = at_v[pl.ds(i * 16, 16)]
        evv = plsc.load_gather(em_v, [tv])
        ev_v[pl.ds(i * 16, 16)] = evv
        for e in range(E):
            cnt = plsc.all_reduce_population_count(evv == e)
            lc = jnp.where(lane == e, lc + cnt, lc)
        return lc
    lc = lax.fori_loop(0, TPT // 16, _hist, jnp.zeros((E,), jnp.int32))
    lc_v[...] = lc
    pltpu.sync_copy(ev_v, ev_hbm.at[pl.ds(wid * TPT, TPT)])
    pltpu.sync_copy(lc_v, lc_hbm.at[wid])


def _ids(em, atom_types):
    f = functools.partial(
        pl.kernel,
        out_type=[jax.ShapeDtypeStruct((N_TOK,), jnp.int32),
                  jax.ShapeDtypeStruct((NW, E), jnp.int32)],
        mesh=_mesh(),
        scratch_types=[
            pltpu.VMEM((NTYPES,), jnp.int32),
            pltpu.VMEM((TPT,), jnp.int32),
            pltpu.VMEM((TPT,), jnp.int32),
            pltpu.VMEM((E,), jnp.int32),
        ],
        compiler_params=_SC_PARAMS,
    )(_ids_body)
    return f(em, atom_types)


def _sort_body(ev_hbm, lc_hbm, x_hbm, pos_hbm, bexp_hbm, xs_hbm,
               ev_v, lc_v, pos2_v, bexp_v, xv_v, sem):
    wid = _wid()
    lane = lax.broadcasted_iota(jnp.int32, (E,), 0)
    pltpu.sync_copy(lc_hbm, lc_v)
    pltpu.sync_copy(ev_hbm.at[pl.ds(wid * TPT, TPT)], ev_v)
    pltpu.sync_copy(x_hbm.at[pl.ds(wid * TPT, TPT)], xv_v)

    g = jnp.zeros((E,), jnp.int32)
    bef = jnp.zeros((E,), jnp.int32)
    for r in range(NW):
        row = lc_v[r, :]
        g = g + row
        bef = bef + jnp.where(r < wid, row, jnp.zeros((E,), jnp.int32))
    padded = ((g + (TM - 1)) >> 7) << 7
    offs = plsc.cumsum(padded) - padded
    start = offs + bef

    @pl.when(wid == 0)
    def _():
        for k in range(NB // 16):
            bv = (lane + 16 * k) * TM
            be = jnp.zeros((E,), jnp.int32)
            for e in range(E):
                be = jnp.where((bv >= offs[e]) & (bv < offs[e] + padded[e]),
                               e, be)
            bexp_v[pl.ds(16 * k, 16)] = be
        pltpu.sync_copy(bexp_v, bexp_hbm)

    for e in range(E):
        def _rank(i, s):
            evv = ev_v[pl.ds(i * 16, 16)]
            m = evv == e
            ones = jnp.where(m, jnp.ones((E,), jnp.int32),
                             jnp.zeros((E,), jnp.int32))
            rk = plsc.cumsum(ones)
            posv = s + rk - 1
            li = i * 16 + lane
            plsc.store_scatter(pos2_v, [li >> 7, li & 127], posv, mask=m)
            return s + plsc.all_reduce_population_count(m)
        lax.fori_loop(0, TPT // 16, _rank,
                      jnp.full((E,), start[e], jnp.int32))

    pltpu.sync_copy(pos2_v, pos_hbm.at[pl.ds(wid * RPT, RPT)])
    for j in range(RPT):
        pltpu.async_copy(xv_v.at[pl.ds(j * 128, 128)],
                         xs_hbm.at[pos2_v.at[j]], sem).wait()


def _sort_scatter(ev, lc, x):
    f = functools.partial(
        pl.kernel,
        out_type=[jax.ShapeDtypeStruct((N_TOK // 128, 128), jnp.int32),
                  jax.ShapeDtypeStruct((NB,), jnp.int32),
                  jax.ShapeDtypeStruct((CAP, NUM_IN), jnp.float32)],
        mesh=_mesh(),
        scratch_types=[
            pltpu.VMEM((TPT,), jnp.int32),
            pltpu.VMEM((NW, E), jnp.int32),
            pltpu.VMEM((RPT, 128), jnp.int32),
            pltpu.VMEM((NB,), jnp.int32),
            pltpu.VMEM((TPT, NUM_IN), jnp.float32),
            pltpu.SemaphoreType.DMA,
        ],
        compiler_params=_SC_PARAMS,
    )(_sort_body)
    return f(ev, lc, x)


def _gemm_body(bexp_ref, xs_ref, w1_ref, w2_ref, b_ref, y1_ref, y2_ref):
    xb = xs_ref[...]
    e = bexp_ref[pl.program_id(0)]
    brow = b_ref[pl.ds(e, 1), :]
    y1_ref[...] = jnp.tanh(
        jnp.dot(xb, w1_ref[0], preferred_element_type=jnp.float32)
        + brow[:, :HALF_OUT])
    y2_ref[...] = jnp.tanh(
        jnp.dot(xb, w2_ref[0], preferred_element_type=jnp.float32)
        + brow[:, HALF_OUT:])


def _gemm(bexp, xs, expert_W, expert_b):
    grid_spec = pltpu.PrefetchScalarGridSpec(
        num_scalar_prefetch=1,
        grid=(NB,),
        in_specs=[
            pl.BlockSpec((TM, NUM_IN), lambda b, s: (b, 0)),
            pl.BlockSpec((1, NUM_IN, HALF_OUT), lambda b, s: (s[b], 0, 0)),
            pl.BlockSpec((1, NUM_IN, HALF_OUT), lambda b, s: (s[b], 0, 1)),
            pl.BlockSpec((E, TOTAL_OUT), lambda b, s: (0, 0)),
        ],
        out_specs=[
            pl.BlockSpec((TM, HALF_OUT), lambda b, s: (b, 0)),
            pl.BlockSpec((TM, HALF_OUT), lambda b, s: (b, 0)),
        ],
    )
    return pl.pallas_call(
        _gemm_body,
        grid_spec=grid_spec,
        out_shape=[jax.ShapeDtypeStruct((CAP, HALF_OUT), jnp.float32),
                   jax.ShapeDtypeStruct((CAP, HALF_OUT), jnp.float32)],
    )(bexp, xs, expert_W, expert_W, expert_b)


def _gath_body(pos_hbm, y1_hbm, y2_hbm, o1_hbm, o2_hbm,
               pv_v, y1_v, y2_v, sem):
    wid = _wid()
    pltpu.sync_copy(pos_hbm.at[pl.ds(wid * RPT, RPT)], pv_v)
    for j in range(RPT):
        pltpu.async_copy(y1_hbm.at[pv_v.at[j]],
                         y1_v.at[pl.ds(j * 128, 128)], sem).wait()
        pltpu.async_copy(y2_hbm.at[pv_v.at[j]],
                         y2_v.at[pl.ds(j * 128, 128)], sem).wait()
    pltpu.sync_copy(y1_v, o1_hbm.at[pl.ds(wid * TPT, TPT)])
    pltpu.sync_copy(y2_v, o2_hbm.at[pl.ds(wid * TPT, TPT)])


def _gather_y(pos, y1, y2):
    f = functools.partial(
        pl.kernel,
        out_type=[jax.ShapeDtypeStruct((N_TOK, HALF_OUT), jnp.float32),
                  jax.ShapeDtypeStruct((N_TOK, HALF_OUT), jnp.float32)],
        mesh=_mesh(),
        scratch_types=[
            pltpu.VMEM((RPT, 128), jnp.int32),
            pltpu.VMEM((TPT, HALF_OUT), jnp.float32),
            pltpu.VMEM((TPT, HALF_OUT), jnp.float32),
            pltpu.SemaphoreType.DMA,
        ],
        compiler_params=_SC_PARAMS,
    )(_gath_body)
    return f(pos, y1, y2)


def kernel(x, type_embeddings, atom_types, gate_W, gate_b, expert_W, expert_b):
    atom_types = atom_types.astype(jnp.int32)
    em = _expert_map(type_embeddings, gate_W, gate_b)
    ev, lc = _ids(em, atom_types)
    pos, bexp, xs = _sort_scatter(ev, lc, x)
    y1, y2 = _gemm(bexp, xs, expert_W, expert_b)
    o1, o2 = _gather_y(pos, y1, y2)
    return (o1, o2)


# speedup vs baseline: 1.7496x; 1.0389x over previous
import functools

import jax
import jax.numpy as jnp
from jax import lax
from jax.experimental import pallas as pl
from jax.experimental.pallas import tpu as pltpu
from jax.experimental.pallas import tpu_sc as plsc

N_TOK = 8192
NUM_IN = 256
TOTAL_OUT = 256
HALF_OUT = 128
E = 16
NTYPES = 128
TEBD = 64

TM = 128
CAP = 10240
NB = CAP // TM

NC, NS = 2, 16
NW = NC * NS
TPT = N_TOK // NW
RPT = TPT // 128
NV = N_TOK // 16


def _wid():
    return lax.axis_index("s") * NC + lax.axis_index("c")


def _mesh():
    return plsc.VectorSubcoreMesh(core_axis_name="c", subcore_axis_name="s")


_SC_PARAMS = pltpu.CompilerParams(needs_layout_passes=False)


def _emap_body(te_ref, gw_ref, gb_ref, em_ref):
    logits = jnp.dot(te_ref[...], gw_ref[...],
                     preferred_element_type=jnp.float32) + gb_ref[...]
    em_ref[...] = jnp.argmax(logits, axis=1).astype(jnp.int32)[None, :]


def _expert_map(type_embeddings, gate_W, gate_b):
    return pl.pallas_call(
        _emap_body,
        out_shape=jax.ShapeDtypeStruct((1, NTYPES), jnp.int32),
    )(type_embeddings, gate_W, gate_b.reshape(1, E))


def _sort_body(em_hbm, at_hbm, x_hbm, pos_hbm, bexp_hbm, xs_hbm,
               em_v, at_v, ev_v, hist_v, pos2_v, bexp_v, xv_v, sem, xsem):
    wid = _wid()
    lane = lax.broadcasted_iota(jnp.int32, (E,), 0)
    ones = jnp.ones((E,), jnp.int32)

    xload = pltpu.async_copy(x_hbm.at[pl.ds(wid * TPT, TPT)], xv_v, xsem)
    pltpu.sync_copy(em_hbm.at[0], em_v)
    pltpu.sync_copy(at_hbm, at_v)

    hist_v[...] = jnp.zeros((E,), jnp.int32)

    def _acc(i, carry):
        tv = at_v[pl.ds(i * 16, 16)]
        evv = plsc.load_gather(em_v, [tv])
        plsc.addupdate_scatter(hist_v, [evv], ones)
        return carry

    lax.fori_loop(0, wid * (TPT // 16), _acc, 0)
    bef = hist_v[...]
    base = wid * (TPT // 16)

    def _acc_own(i, carry):
        tv = at_v[pl.ds((base + i) * 16, 16)]
        evv = plsc.load_gather(em_v, [tv])
        ev_v[pl.ds(i * 16, 16)] = evv
        plsc.addupdate_scatter(hist_v, [evv], ones)
        return carry
    lax.fori_loop(0, TPT // 16, _acc_own, 0)
    lax.fori_loop(base + TPT // 16, NV, _acc, 0)
    g = hist_v[...]

    padded = ((g + (TM - 1)) >> 7) << 7
    offs = plsc.cumsum(padded) - padded
    start = offs + bef

    @pl.when(wid == 0)
    def _():
        for k in range(NB // 16):
            bv = (lane + 16 * k) * TM
            be = jnp.zeros((E,), jnp.int32)
            for e in range(E):
                be = jnp.where((bv >= offs[e]) & (bv < offs[e] + padded[e]),
                               e, be)
            bexp_v[pl.ds(16 * k, 16)] = be
        pltpu.sync_copy(bexp_v, bexp_hbm)

    for e in range(E):
        def _rank(i, s):
            evv = ev_v[pl.ds(i * 16, 16)]
            m = evv == e
            rk = plsc.cumsum(jnp.where(m, ones, jnp.zeros((E,), jnp.int32)))
            posv = jnp.minimum(s + rk - 1, CAP - 1)
            li = i * 16 + lane
            plsc.store_scatter(pos2_v, [li >> 7, li & 127], posv, mask=m)
            return s + plsc.all_reduce_population_count(m)
        lax.fori_loop(0, TPT // 16, _rank,
                      jnp.full((E,), start[e], jnp.int32))

    pltpu.sync_copy(pos2_v, pos_hbm.at[pl.ds(wid * RPT, RPT)])
    xload.wait()
    cps = [pltpu.async_copy(xv_v.at[pl.ds(j * 128, 128)],
                            xs_hbm.at[pos2_v.at[j]], sem)
           for j in range(RPT)]
    for c in cps:
        c.wait()


def _sort_scatter(em, atom_types, x):
    f = functools.partial(
        pl.kernel,
        out_type=[jax.ShapeDtypeStruct((N_TOK // 128, 128), jnp.int32),
                  jax.ShapeDtypeStruct((NB,), jnp.int32),
                  jax.ShapeDtypeStruct((CAP, NUM_IN), jnp.float32)],
        mesh=_mesh(),
        scratch_types=[
            pltpu.VMEM((NTYPES,), jnp.int32),
            pltpu.VMEM((N_TOK,), jnp.int32),
            pltpu.VMEM((TPT,), jnp.int32),
            pltpu.VMEM((E,), jnp.int32),
            pltpu.VMEM((RPT, 128), jnp.int32),
            pltpu.VMEM((NB,), jnp.int32),
            pltpu.VMEM((TPT, NUM_IN), jnp.float32),
            pltpu.SemaphoreType.DMA,
            pltpu.SemaphoreType.DMA,
        ],
        compiler_params=_SC_PARAMS,
    )(_sort_body)
    return f(em, atom_types, x)


def _gemm_body(bexp_ref, xs_ref, w1_ref, w2_ref, b_ref, y1_ref, y2_ref):
    xb = xs_ref[...]
    e = bexp_ref[pl.program_id(0)]
    brow = b_ref[pl.ds(e, 1), :]
    y1_ref[...] = jnp.tanh(
        jnp.dot(xb, w1_ref[0], preferred_element_type=jnp.float32)
        + brow[:, :HALF_OUT])
    y2_ref[...] = jnp.tanh(
        jnp.dot(xb, w2_ref[0], preferred_element_type=jnp.float32)
        + brow[:, HALF_OUT:])


def _gemm(bexp, xs, expert_W, expert_b):
    grid_spec = pltpu.PrefetchScalarGridSpec(
        num_scalar_prefetch=1,
        grid=(NB,),
        in_specs=[
            pl.BlockSpec((TM, NUM_IN), lambda b, s: (b, 0)),
            pl.BlockSpec((1, NUM_IN, HALF_OUT), lambda b, s: (s[b], 0, 0)),
            pl.BlockSpec((1, NUM_IN, HALF_OUT), lambda b, s: (s[b], 0, 1)),
            pl.BlockSpec((E, TOTAL_OUT), lambda b, s: (0, 0)),
        ],
        out_specs=[
            pl.BlockSpec((TM, HALF_OUT), lambda b, s: (b, 0)),
            pl.BlockSpec((TM, HALF_OUT), lambda b, s: (b, 0)),
        ],
    )
    return pl.pallas_call(
        _gemm_body,
        grid_spec=grid_spec,
        out_shape=[jax.ShapeDtypeStruct((CAP, HALF_OUT), jnp.float32),
                   jax.ShapeDtypeStruct((CAP, HALF_OUT), jnp.float32)],
    )(bexp, xs, expert_W, expert_W, expert_b)


def _gath_body(pos_hbm, y1_hbm, y2_hbm, o1_hbm, o2_hbm,
               pv_v, y1_v, y2_v, sem):
    wid = _wid()
    pltpu.sync_copy(pos_hbm.at[pl.ds(wid * RPT, RPT)], pv_v)
    cps = []
    for j in range(RPT):
        cps.append(pltpu.async_copy(y1_hbm.at[pv_v.at[j]],
                                    y1_v.at[pl.ds(j * 128, 128)], sem))
        cps.append(pltpu.async_copy(y2_hbm.at[pv_v.at[j]],
                                    y2_v.at[pl.ds(j * 128, 128)], sem))
    for c in cps:
        c.wait()
    pltpu.sync_copy(y1_v, o1_hbm.at[pl.ds(wid * TPT, TPT)])
    pltpu.sync_copy(y2_v, o2_hbm.at[pl.ds(wid * TPT, TPT)])


def _gather_y(pos, y1, y2):
    f = functools.partial(
        pl.kernel,
        out_type=[jax.ShapeDtypeStruct((N_TOK, HALF_OUT), jnp.float32),
                  jax.ShapeDtypeStruct((N_TOK, HALF_OUT), jnp.float32)],
        mesh=_mesh(),
        scratch_types=[
            pltpu.VMEM((RPT, 128), jnp.int32),
            pltpu.VMEM((TPT, HALF_OUT), jnp.float32),
            pltpu.VMEM((TPT, HALF_OUT), jnp.float32),
            pltpu.SemaphoreType.DMA,
        ],
        compiler_params=_SC_PARAMS,
    )(_gath_body)
    return f(pos, y1, y2)


def kernel(x, type_embeddings, atom_types, gate_W, gate_b, expert_W, expert_b):
    atom_types = atom_types.astype(jnp.int32)
    em = _expert_map(type_embeddings, gate_W, gate_b)
    pos, bexp, xs = _sort_scatter(em, atom_types, x)
    y1, y2 = _gemm(bexp, xs, expert_W, expert_b)
    o1, o2 = _gather_y(pos, y1, y2)
    return (o1, o2)


# speedup vs baseline: 2.0887x; 1.1938x over previous
import functools

import jax
import jax.numpy as jnp
from jax import lax
from jax.experimental import pallas as pl
from jax.experimental.pallas import tpu as pltpu
from jax.experimental.pallas import tpu_sc as plsc

N_TOK = 8192
NUM_IN = 256
TOTAL_OUT = 256
HALF_OUT = 128
E = 16
NTYPES = 128
TEBD = 64

TM = 256
CAP = 12288
NB = CAP // TM

NC, NS = 2, 16
NW = NC * NS
TPT = N_TOK // NW
RPT = TPT // 128
NV = N_TOK // 16


def _wid():
    return lax.axis_index("s") * NC + lax.axis_index("c")


def _mesh():
    return plsc.VectorSubcoreMesh(core_axis_name="c", subcore_axis_name="s")


_SC_PARAMS = pltpu.CompilerParams(needs_layout_passes=False)


def _emap_body(te_ref, gw_ref, gb_ref, em_ref):
    logits = jnp.dot(te_ref[...], gw_ref[...],
                     preferred_element_type=jnp.float32) + gb_ref[...]
    em_ref[...] = jnp.argmax(logits, axis=1).astype(jnp.int32)[None, :]


def _expert_map(type_embeddings, gate_W, gate_b):
    return pl.pallas_call(
        _emap_body,
        out_shape=jax.ShapeDtypeStruct((1, NTYPES), jnp.int32),
    )(type_embeddings, gate_W, gate_b.reshape(1, E))


def _sort_body(em_hbm, at_hbm, x_hbm, pos_hbm, bexp_hbm, xs_hbm,
               em_v, at_v, ev_v, hist_v, pos2_v, bexp_v, xv_v, sem, xsem):
    wid = _wid()
    lane = lax.broadcasted_iota(jnp.int32, (E,), 0)
    ones = jnp.ones((E,), jnp.int32)

    xload = pltpu.async_copy(x_hbm.at[pl.ds(wid * TPT, TPT)], xv_v, xsem)
    pltpu.sync_copy(em_hbm.at[0], em_v)
    pltpu.sync_copy(at_hbm, at_v)

    hist_v[...] = jnp.zeros((E,), jnp.int32)

    def _acc(i, carry):
        tv = at_v[pl.ds(i * 16, 16)]
        evv = plsc.load_gather(em_v, [tv])
        plsc.addupdate_scatter(hist_v, [evv], ones)
        return carry

    lax.fori_loop(0, wid * (TPT // 16), _acc, 0)
    bef = hist_v[...]
    base = wid * (TPT // 16)

    def _acc_own(i, carry):
        tv = at_v[pl.ds((base + i) * 16, 16)]
        evv = plsc.load_gather(em_v, [tv])
        ev_v[pl.ds(i * 16, 16)] = evv
        plsc.addupdate_scatter(hist_v, [evv], ones)
        return carry
    lax.fori_loop(0, TPT // 16, _acc_own, 0)
    lax.fori_loop(base + TPT // 16, NV, _acc, 0)
    g = hist_v[...]

    padded = ((g + (TM - 1)) >> 7) << 7
    offs = plsc.cumsum(padded) - padded
    start = offs + bef

    @pl.when(wid == 0)
    def _():
        for k in range(NB // 16):
            bv = (lane + 16 * k) * TM
            be = jnp.zeros((E,), jnp.int32)
            for e in range(E):
                be = jnp.where((bv >= offs[e]) & (bv < offs[e] + padded[e]),
                               e, be)
            bexp_v[pl.ds(16 * k, 16)] = be
        pltpu.sync_copy(bexp_v, bexp_hbm)

    for e in range(E):
        def _rank(i, s):
            evv = ev_v[pl.ds(i * 16, 16)]
            m = evv == e
            rk = plsc.cumsum(jnp.where(m, ones, jnp.zeros((E,), jnp.int32)))
            posv = jnp.minimum(s + rk - 1, CAP - 1)
            li = i * 16 + lane
            plsc.store_scatter(pos2_v, [li >> 7, li & 127], posv, mask=m)
            return s + plsc.all_reduce_population_count(m)
        lax.fori_loop(0, TPT // 16, _rank,
                      jnp.full((E,), start[e], jnp.int32))

    pltpu.sync_copy(pos2_v, pos_hbm.at[pl.ds(wid * RPT, RPT)])
    xload.wait()
    cps = [pltpu.async_copy(xv_v.at[pl.ds(j * 128, 128)],
                            xs_hbm.at[pos2_v.at[j]], sem)
           for j in range(RPT)]
    for c in cps:
        c.wait()


def _sort_scatter(em, atom_types, x):
    f = functools.partial(
        pl.kernel,
        out_type=[jax.ShapeDtypeStruct((N_TOK // 128, 128), jnp.int32),
                  jax.ShapeDtypeStruct((NB,), jnp.int32),
                  jax.ShapeDtypeStruct((CAP, NUM_IN), jnp.float32)],
        mesh=_mesh(),
        scratch_types=[
            pltpu.VMEM((NTYPES,), jnp.int32),
            pltpu.VMEM((N_TOK,), jnp.int32),
            pltpu.VMEM((TPT,), jnp.int32),
            pltpu.VMEM((E,), jnp.int32),
            pltpu.VMEM((RPT, 128), jnp.int32),
            pltpu.VMEM((NB,), jnp.int32),
            pltpu.VMEM((TPT, NUM_IN), jnp.float32),
            pltpu.SemaphoreType.DMA,
            pltpu.SemaphoreType.DMA,
        ],
        compiler_params=_SC_PARAMS,
    )(_sort_body)
    return f(em, atom_types, x)


def _gemm_body(bexp_ref, xs_ref, w_ref, b_ref, y1_ref, y2_ref):
    xb = xs_ref[...].astype(jnp.bfloat16)
    e = bexp_ref[pl.program_id(0)]
    brow = b_ref[pl.ds(e, 1), :]
    w = w_ref[0].astype(jnp.bfloat16)
    y = jnp.tanh(
        jnp.dot(xb, w, preferred_element_type=jnp.float32) + brow)
    y1_ref[...] = y[:, :HALF_OUT]
    y2_ref[...] = y[:, HALF_OUT:]


def _gemm(bexp, xs, expert_W, expert_b):
    grid_spec = pltpu.PrefetchScalarGridSpec(
        num_scalar_prefetch=1,
        grid=(NB,),
        in_specs=[
            pl.BlockSpec((TM, NUM_IN), lambda b, s: (b, 0)),
            pl.BlockSpec((1, NUM_IN, TOTAL_OUT), lambda b, s: (s[b], 0, 0)),
            pl.BlockSpec((E, TOTAL_OUT), lambda b, s: (0, 0)),
        ],
        out_specs=[
            pl.BlockSpec((TM, HALF_OUT), lambda b, s: (b, 0)),
            pl.BlockSpec((TM, HALF_OUT), lambda b, s: (b, 0)),
        ],
    )
    return pl.pallas_call(
        _gemm_body,
        grid_spec=grid_spec,
        out_shape=[jax.ShapeDtypeStruct((CAP, HALF_OUT), jnp.float32),
                   jax.ShapeDtypeStruct((CAP, HALF_OUT), jnp.float32)],
    )(bexp, xs, expert_W, expert_b)


def _gath_body(pos_hbm, y1_hbm, y2_hbm, o1_hbm, o2_hbm,
               pv_v, y1_v, y2_v, sem):
    wid = _wid()
    pltpu.sync_copy(pos_hbm.at[pl.ds(wid * RPT, RPT)], pv_v)
    cps = []
    for j in range(RPT):
        cps.append(pltpu.async_copy(y1_hbm.at[pv_v.at[j]],
                                    y1_v.at[pl.ds(j * 128, 128)], sem))
        cps.append(pltpu.async_copy(y2_hbm.at[pv_v.at[j]],
                                    y2_v.at[pl.ds(j * 128, 128)], sem))
    for c in cps:
        c.wait()
    pltpu.sync_copy(y1_v, o1_hbm.at[pl.ds(wid * TPT, TPT)])
    pltpu.sync_copy(y2_v, o2_hbm.at[pl.ds(wid * TPT, TPT)])


def _gather_y(pos, y1, y2):
    f = functools.partial(
        pl.kernel,
        out_type=[jax.ShapeDtypeStruct((N_TOK, HALF_OUT), jnp.float32),
                  jax.ShapeDtypeStruct((N_TOK, HALF_OUT), jnp.float32)],
        mesh=_mesh(),
        scratch_types=[
            pltpu.VMEM((RPT, 128), jnp.int32),
            pltpu.VMEM((TPT, HALF_OUT), jnp.float32),
            pltpu.VMEM((TPT, HALF_OUT), jnp.float32),
            pltpu.SemaphoreType.DMA,
        ],
        compiler_params=_SC_PARAMS,
    )(_gath_body)
    return f(pos, y1, y2)


def kernel(x, type_embeddings, atom_types, gate_W, gate_b, expert_W, expert_b):
    atom_types = atom_types.astype(jnp.int32)
    em = _expert_map(type_embeddings, gate_W, gate_b)
    pos, bexp, xs = _sort_scatter(em, atom_types, x)
    y1, y2 = _gemm(bexp, xs, expert_W, expert_b)
    o1, o2 = _gather_y(pos, y1, y2)
    return (o1, o2)
